# baseline (speedup 1.0000x reference)
import functools

import jax
import jax.numpy as jnp
from jax import lax
from jax.experimental import pallas as pl
from jax.experimental.pallas import tpu as pltpu
from jax.experimental.pallas import tpu_sc as plsc


def kernel(x, embedding_table):
    seq_len = x.shape[1]
    table_rows, d_model = embedding_table.shape

    info = plsc.get_sparse_core_info()
    nc, ns = info.num_cores, info.num_subcores
    nw = nc * ns
    rows_per_w = seq_len // nw
    row_elems = seq_len * d_model

    mesh = plsc.VectorSubcoreMesh(core_axis_name="c", subcore_axis_name="s")

    @functools.partial(
        pl.kernel,
        mesh=mesh,
        out_type=jax.ShapeDtypeStruct((seq_len * seq_len * d_model,), jnp.float32),
        scratch_types=[
            pltpu.VMEM_SHARED((2 * seq_len * d_model,), jnp.float32),
        ],
    )
    def copy_kernel(table_hbm, out_hbm, spmem):
        cid = lax.axis_index("c")
        sid = lax.axis_index("s")
        wid = sid * nc + cid

        @pl.when(sid == 0)
        def _stage():
            pltpu.sync_copy(table_hbm.at[pl.ds(0, 2 * seq_len * d_model)], spmem)

        plsc.subcore_barrier()
        for r in range(rows_per_w):
            i = wid * rows_per_w + r
            src = pl.multiple_of((seq_len - i) * d_model, 128)
            dst = pl.multiple_of(i * row_elems, 128)
            pltpu.sync_copy(
                spmem.at[pl.ds(src, row_elems)],
                out_hbm.at[pl.ds(dst, row_elems)],
            )

    flat = copy_kernel(embedding_table.reshape(-1))
    return flat.reshape(seq_len, seq_len, d_model)


# speedup vs baseline: 1.0055x; 1.0055x over previous
import functools

import jax
import jax.numpy as jnp
from jax import lax
from jax.experimental import pallas as pl
from jax.experimental.pallas import tpu as pltpu
from jax.experimental.pallas import tpu_sc as plsc


def kernel(x, embedding_table):
    seq_len = x.shape[1]
    table_rows, d_model = embedding_table.shape

    info = plsc.get_sparse_core_info()
    nc, ns = info.num_cores, info.num_subcores
    nw = nc * ns
    rows_per_w = seq_len // nw
    row_elems = seq_len * d_model

    mesh = plsc.VectorSubcoreMesh(core_axis_name="c", subcore_axis_name="s")

    @functools.partial(
        pl.kernel,
        mesh=mesh,
        out_type=jax.ShapeDtypeStruct((seq_len * seq_len * d_model,), jnp.float32),
        scratch_types=[
            pltpu.VMEM_SHARED((2 * seq_len * d_model,), jnp.float32),
            pltpu.SemaphoreType.DMA,
        ],
    )
    def copy_kernel(table_hbm, out_hbm, spmem, sem):
        cid = lax.axis_index("c")
        sid = lax.axis_index("s")
        wid = sid * nc + cid

        @pl.when(sid == 0)
        def _stage():
            pltpu.sync_copy(table_hbm.at[pl.ds(0, 2 * seq_len * d_model)], spmem)

        plsc.subcore_barrier()
        copies = []
        for r in range(rows_per_w):
            i = wid * rows_per_w + r
            src = pl.multiple_of((seq_len - i) * d_model, 128)
            dst = pl.multiple_of(i * row_elems, 128)
            copies.append(
                pltpu.async_copy(
                    spmem.at[pl.ds(src, row_elems)],
                    out_hbm.at[pl.ds(dst, row_elems)],
                    sem,
                )
            )
        for cp in copies:
            cp.wait()

    flat = copy_kernel(embedding_table.reshape(-1))
    return flat.reshape(seq_len, seq_len, d_model)
